# baseline (speedup 1.0000x reference)
import functools

import jax
import jax.numpy as jnp
from jax.experimental import pallas as pl
from jax.experimental.pallas import tpu as pltpu

_SIGN = -2147483648


def _topk_softmax_t(logits, k):
    bits = jax.lax.bitcast_convert_type(logits, jnp.int32)
    key = bits ^ jnp.where(bits < 0, jnp.int32(0x7FFFFFFF), jnp.int32(0))
    cols = logits.shape[1]

    def body(i, t):
        bit = jnp.left_shift(jnp.int32(1), 31 - i)
        t_try = t | bit
        thr = t_try ^ _SIGN
        cnt = jnp.sum((key >= thr).astype(jnp.int32), axis=0, keepdims=True)
        return jnp.where(cnt >= k, t_try, t)

    t = jax.lax.fori_loop(0, 32, body, jnp.zeros((1, cols), jnp.int32))
    mask = key >= (t ^ _SIGN)
    m = jnp.max(logits, axis=0, keepdims=True)
    p = jnp.where(mask, jnp.exp(logits - m), 0.0)
    return p / jnp.sum(p, axis=0, keepdims=True)


def _gating_kernel(vals_ref, alpha_ref, temp_ref, u_ref, v_ref, lnw_ref,
                   lnb_ref, out_ref, attn_alpha_scr, *, k, scale):
    h = pl.program_id(0)
    b = pl.program_id(1)
    vals = vals_ref[0, 0]

    energy = jnp.mean(vals * vals, axis=0, keepdims=True)
    rms = jnp.maximum(jnp.sqrt(jnp.mean(energy)), 1e-6)
    gain = jax.nn.softplus(temp_ref[h, 0])
    score = energy * (gain / rms)
    mu = jnp.mean(score)
    var = jnp.mean((score - mu) ** 2)
    score = (score - mu) * jax.lax.rsqrt(var + 1e-5)
    score = score * lnw_ref[:, :] + lnb_ref[:, :]
    score_col = jnp.transpose(score)
    bilinear = jnp.dot(v_ref[0], u_ref[0],
                       preferred_element_type=jnp.float32)
    logits = bilinear + score_col

    attn = _topk_softmax_t(logits, k)

    @pl.when(b == 0)
    def _():
        attn_alpha_scr[:, :] = _topk_softmax_t(alpha_ref[0] * scale, k)

    attn = attn + attn_alpha_scr[:, :]
    out_ref[0, 0] = jnp.dot(vals, attn, preferred_element_type=jnp.float32)


def kernel(values, alpha, temp, gamma_hs, U, V, ln_w, ln_b):
    del gamma_hs
    B, F, H, D = values.shape
    _, S, _ = alpha.shape
    R = U.shape[-1]
    k = max(1, int(0.1 * F))
    scale = 1.0 / (F ** 0.5)

    vt = jnp.transpose(values, (0, 2, 3, 1))
    alpha_t = jnp.transpose(alpha, (0, 2, 1))
    u_t = jnp.transpose(U, (0, 2, 1))
    v_t = jnp.transpose(V, (0, 2, 1))

    grid = (H, B)
    out = pl.pallas_call(
        functools.partial(_gating_kernel, k=k, scale=scale),
        grid=grid,
        in_specs=[
            pl.BlockSpec((1, 1, D, F), lambda h, b: (b, h, 0, 0)),
            pl.BlockSpec((1, F, S), lambda h, b: (h, 0, 0)),
            pl.BlockSpec(memory_space=pltpu.SMEM),
            pl.BlockSpec((1, R, S), lambda h, b: (h, 0, 0)),
            pl.BlockSpec((1, F, R), lambda h, b: (h, 0, 0)),
            pl.BlockSpec((1, F), lambda h, b: (0, 0)),
            pl.BlockSpec((1, F), lambda h, b: (0, 0)),
        ],
        out_specs=pl.BlockSpec((1, 1, D, S), lambda h, b: (b, h, 0, 0)),
        out_shape=jax.ShapeDtypeStruct((B, H, D, S), jnp.float32),
        scratch_shapes=[pltpu.VMEM((F, S), jnp.float32)],
        compiler_params=pltpu.CompilerParams(
            dimension_semantics=("arbitrary", "arbitrary"),
        ),
    )(vt, alpha_t, temp.astype(jnp.float32), u_t, v_t,
      ln_w.reshape(1, F), ln_b.reshape(1, F))
    return jnp.transpose(out, (0, 3, 1, 2))


# speedup vs baseline: 1.2771x; 1.2771x over previous
import functools

import jax
import jax.numpy as jnp
from jax.experimental import pallas as pl
from jax.experimental.pallas import tpu as pltpu

_SIGN = -2147483648


def _count_ge16(key16, thr):
    f, cols = key16.shape
    one = jnp.int16(1)
    zero = jnp.int16(0)
    acc = jnp.zeros((16, cols), jnp.int16)
    for j in range(f // 16):
        acc = acc + jnp.where(key16[16 * j:16 * (j + 1)] >= thr, one, zero)
    return jnp.sum(acc.astype(jnp.int32), axis=0, keepdims=True)


def _count_gt16(key16, thr):
    f, cols = key16.shape
    one = jnp.int16(1)
    zero = jnp.int16(0)
    acc = jnp.zeros((16, cols), jnp.int16)
    for j in range(f // 16):
        acc = acc + jnp.where(key16[16 * j:16 * (j + 1)] > thr, one, zero)
    return jnp.sum(acc.astype(jnp.int32), axis=0, keepdims=True)


def _radix16(key16, target, cols):
    def body(i, t):
        bit = jnp.left_shift(jnp.int32(1), 15 - i)
        t_try = t | bit
        thr = (t_try ^ 0x8000).astype(jnp.int16)
        cnt = _count_ge16(key16, thr)
        return jnp.where(cnt >= target, t_try, t)

    t = jax.lax.fori_loop(0, 16, body, jnp.zeros((1, cols), jnp.int32))
    return (t ^ 0x8000).astype(jnp.int16)


def _topk_softmax_t(logits, k):
    bits = jax.lax.bitcast_convert_type(logits, jnp.int32)
    key = bits ^ jnp.where(bits < 0, jnp.int32(0x7FFFFFFF), jnp.int32(0))
    cols = logits.shape[1]

    khi = jax.lax.shift_right_arithmetic(key, 16).astype(jnp.int16)
    t_hi = _radix16(khi, jnp.int32(k), cols)

    c_gt = _count_gt16(khi, t_hi)
    border = khi == t_hi
    klo = (key.astype(jnp.int16)) ^ jnp.int16(-32768)
    klo_m = jnp.where(border, klo, jnp.int16(-32768))
    t_lo = _radix16(klo_m, jnp.int32(k) - c_gt, cols)

    mask = (khi > t_hi) | (border & (klo_m >= t_lo))
    m = jnp.max(logits, axis=0, keepdims=True)
    p = jnp.where(mask, jnp.exp(logits - m), 0.0)
    return p / jnp.sum(p, axis=0, keepdims=True)


def _gating_kernel(vals_ref, alpha_ref, temp_ref, u_ref, v_ref, lnw_ref,
                   lnb_ref, out_ref, attn_alpha_scr, *, k, scale):
    h = pl.program_id(0)
    b = pl.program_id(1)
    vals = vals_ref[0, 0]

    energy = jnp.mean(vals * vals, axis=0, keepdims=True)
    rms = jnp.maximum(jnp.sqrt(jnp.mean(energy)), 1e-6)
    gain = jax.nn.softplus(temp_ref[h, 0])
    score = energy * (gain / rms)
    mu = jnp.mean(score)
    var = jnp.mean((score - mu) ** 2)
    score = (score - mu) * jax.lax.rsqrt(var + 1e-5)
    score = score * lnw_ref[:, :] + lnb_ref[:, :]
    score_col = jnp.transpose(score)
    bilinear = jnp.dot(v_ref[0], u_ref[0],
                       preferred_element_type=jnp.float32)
    logits = bilinear + score_col

    attn = _topk_softmax_t(logits, k)

    @pl.when(b == 0)
    def _():
        attn_alpha_scr[:, :] = _topk_softmax_t(alpha_ref[0] * scale, k)

    attn = attn + attn_alpha_scr[:, :]
    out_ref[0, 0] = jnp.dot(vals, attn, preferred_element_type=jnp.float32)


def kernel(values, alpha, temp, gamma_hs, U, V, ln_w, ln_b):
    del gamma_hs
    B, F, H, D = values.shape
    _, S, _ = alpha.shape
    R = U.shape[-1]
    k = max(1, int(0.1 * F))
    scale = 1.0 / (F ** 0.5)

    vt = jnp.transpose(values, (0, 2, 3, 1))
    alpha_t = jnp.transpose(alpha, (0, 2, 1))
    u_t = jnp.transpose(U, (0, 2, 1))
    v_t = jnp.transpose(V, (0, 2, 1))

    grid = (H, B)
    out = pl.pallas_call(
        functools.partial(_gating_kernel, k=k, scale=scale),
        grid=grid,
        in_specs=[
            pl.BlockSpec((1, 1, D, F), lambda h, b: (b, h, 0, 0)),
            pl.BlockSpec((1, F, S), lambda h, b: (h, 0, 0)),
            pl.BlockSpec(memory_space=pltpu.SMEM),
            pl.BlockSpec((1, R, S), lambda h, b: (h, 0, 0)),
            pl.BlockSpec((1, F, R), lambda h, b: (h, 0, 0)),
            pl.BlockSpec((1, F), lambda h, b: (0, 0)),
            pl.BlockSpec((1, F), lambda h, b: (0, 0)),
        ],
        out_specs=pl.BlockSpec((1, 1, D, S), lambda h, b: (b, h, 0, 0)),
        out_shape=jax.ShapeDtypeStruct((B, H, D, S), jnp.float32),
        scratch_shapes=[pltpu.VMEM((F, S), jnp.float32)],
        compiler_params=pltpu.CompilerParams(
            dimension_semantics=("arbitrary", "arbitrary"),
        ),
    )(vt, alpha_t, temp.astype(jnp.float32), u_t, v_t,
      ln_w.reshape(1, F), ln_b.reshape(1, F))
    return jnp.transpose(out, (0, 3, 1, 2))


# speedup vs baseline: 1.2797x; 1.0020x over previous
import functools

import jax
import jax.numpy as jnp
from jax.experimental import pallas as pl
from jax.experimental.pallas import tpu as pltpu


def _count_ge16(key16, thr):
    f, cols = key16.shape
    one = jnp.int16(1)
    zero = jnp.int16(0)
    acc = jnp.zeros((16, cols), jnp.int16)
    for j in range(f // 16):
        acc = acc + jnp.where(key16[16 * j:16 * (j + 1)] >= thr, one, zero)
    return jnp.sum(acc.astype(jnp.int32), axis=0, keepdims=True)


def _count_gt16(key16, thr):
    f, cols = key16.shape
    one = jnp.int16(1)
    zero = jnp.int16(0)
    acc = jnp.zeros((16, cols), jnp.int16)
    for j in range(f // 16):
        acc = acc + jnp.where(key16[16 * j:16 * (j + 1)] > thr, one, zero)
    return jnp.sum(acc.astype(jnp.int32), axis=0, keepdims=True)


def _radix16(key16, target, cols):
    def body(i, t):
        bit = jnp.left_shift(jnp.int32(1), 15 - i)
        t_try = t | bit
        thr = (t_try ^ 0x8000).astype(jnp.int16)
        cnt = _count_ge16(key16, thr)
        return jnp.where(cnt >= target, t_try, t)

    t = jax.lax.fori_loop(0, 16, body, jnp.zeros((1, cols), jnp.int32))
    return (t ^ 0x8000).astype(jnp.int16)


def _topk_softmax_t(logits, k):
    bits = jax.lax.bitcast_convert_type(logits, jnp.int32)
    key = bits ^ jnp.where(bits < 0, jnp.int32(0x7FFFFFFF), jnp.int32(0))
    cols = logits.shape[1]

    khi = jax.lax.shift_right_arithmetic(key, 16).astype(jnp.int16)
    t_hi = _radix16(khi, jnp.int32(k), cols)

    c_gt = _count_gt16(khi, t_hi)
    border = khi == t_hi
    klo = (key.astype(jnp.int16)) ^ jnp.int16(-32768)
    klo_m = jnp.where(border, klo, jnp.int16(-32768))
    t_lo = _radix16(klo_m, jnp.int32(k) - c_gt, cols)

    mask = (khi > t_hi) | (border & (klo_m >= t_lo))
    m = jnp.max(logits, axis=0, keepdims=True)
    p = jnp.where(mask, jnp.exp(logits - m), 0.0)
    return p * (1.0 / jnp.sum(p, axis=0, keepdims=True))


def _gating_kernel(vals_ref, alpha_ref, temp_ref, u_ref, v_ref, lnw_ref,
                   lnb_ref, out_ref, attn_alpha_scr, *, k, scale):
    h = pl.program_id(0)
    b = pl.program_id(1)
    vals = vals_ref[0]

    energy = jnp.mean(vals * vals, axis=1, keepdims=True)
    rms = jnp.maximum(jnp.sqrt(jnp.mean(energy)), 1e-6)
    gain = jax.nn.softplus(temp_ref[h, 0])
    score = energy * (gain / rms)
    mu = jnp.mean(score)
    var = jnp.mean((score - mu) ** 2)
    score = (score - mu) * jax.lax.rsqrt(var + 1e-5)
    score = score * lnw_ref[:, :] + lnb_ref[:, :]
    bilinear = jnp.dot(v_ref[0], u_ref[0],
                       preferred_element_type=jnp.float32)
    logits = bilinear + score

    attn = _topk_softmax_t(logits, k)

    @pl.when(b == 0)
    def _():
        alpha_t = jnp.transpose(alpha_ref[0])
        attn_alpha_scr[:, :] = _topk_softmax_t(alpha_t * scale, k)

    attn = attn + attn_alpha_scr[:, :]
    out_ref[0] = jax.lax.dot_general(
        attn, vals, (((0,), (0,)), ((), ())),
        preferred_element_type=jnp.float32)


def kernel(values, alpha, temp, gamma_hs, U, V, ln_w, ln_b):
    del gamma_hs
    B, F, H, D = values.shape
    _, S, _ = alpha.shape
    R = U.shape[-1]
    k = max(1, int(0.1 * F))
    scale = 1.0 / (F ** 0.5)

    vals2 = values.reshape(B, F, H * D)
    u_t = jnp.transpose(U, (0, 2, 1))
    v_t = jnp.transpose(V, (0, 2, 1))

    grid = (H, B)
    out = pl.pallas_call(
        functools.partial(_gating_kernel, k=k, scale=scale),
        grid=grid,
        in_specs=[
            pl.BlockSpec((1, F, D), lambda h, b: (b, 0, h)),
            pl.BlockSpec((1, S, F), lambda h, b: (h, 0, 0)),
            pl.BlockSpec(memory_space=pltpu.SMEM),
            pl.BlockSpec((1, R, S), lambda h, b: (h, 0, 0)),
            pl.BlockSpec((1, F, R), lambda h, b: (h, 0, 0)),
            pl.BlockSpec((F, 1), lambda h, b: (0, 0)),
            pl.BlockSpec((F, 1), lambda h, b: (0, 0)),
        ],
        out_specs=pl.BlockSpec((1, S, D), lambda h, b: (b, 0, h)),
        out_shape=jax.ShapeDtypeStruct((B, S, H * D), jnp.float32),
        scratch_shapes=[pltpu.VMEM((F, S), jnp.float32)],
        compiler_params=pltpu.CompilerParams(
            dimension_semantics=("arbitrary", "arbitrary"),
        ),
    )(vals2, alpha, temp.astype(jnp.float32), u_t, v_t,
      ln_w.reshape(F, 1), ln_b.reshape(F, 1))
    return out.reshape(B, S, H, D)


# speedup vs baseline: 1.4060x; 1.0988x over previous
import functools

import jax
import jax.numpy as jnp
from jax.experimental import pallas as pl
from jax.experimental.pallas import tpu as pltpu

_SIGN = -2147483648


def _count_ge16(key16, thr):
    f, cols = key16.shape
    one = jnp.int16(1)
    zero = jnp.int16(0)
    acc = jnp.zeros((16, cols), jnp.int16)
    for j in range(f // 16):
        acc = acc + jnp.where(key16[16 * j:16 * (j + 1)] >= thr, one, zero)
    return jnp.sum(acc.astype(jnp.int32), axis=0, keepdims=True)


def _count_gt16(key16, thr):
    f, cols = key16.shape
    one = jnp.int16(1)
    zero = jnp.int16(0)
    acc = jnp.zeros((16, cols), jnp.int16)
    for j in range(f // 16):
        acc = acc + jnp.where(key16[16 * j:16 * (j + 1)] > thr, one, zero)
    return jnp.sum(acc.astype(jnp.int32), axis=0, keepdims=True)


def _radix16(key16, target, cols):
    def body(i, t):
        bit = jnp.left_shift(jnp.int32(1), 15 - i)
        t_try = t | bit
        thr = (t_try ^ 0x8000).astype(jnp.int16)
        cnt = _count_ge16(key16, thr)
        return jnp.where(cnt >= target, t_try, t)

    return jax.lax.fori_loop(0, 16, body, jnp.zeros((1, cols), jnp.int32))


def _topk_softmax_t(logits, k):
    bits = jax.lax.bitcast_convert_type(logits, jnp.int32)
    key = bits ^ jnp.where(bits < 0, jnp.int32(0x7FFFFFFF), jnp.int32(0))
    cols = logits.shape[1]

    khi = jax.lax.shift_right_arithmetic(key, 16).astype(jnp.int16)
    t_a = _radix16(khi, jnp.int32(k), cols)
    t_hi = (t_a ^ 0x8000).astype(jnp.int16)

    c_gt = _count_gt16(khi, t_hi)
    border = khi == t_hi
    klo = (key.astype(jnp.int16)) ^ jnp.int16(-32768)
    klo_m = jnp.where(border, klo, jnp.int16(-32768))
    t_b = _radix16(klo_m, jnp.int32(k) - c_gt, cols)

    thr32 = (jnp.left_shift(t_a, 16) | t_b) ^ jnp.int32(_SIGN)
    m = jnp.max(logits, axis=0, keepdims=True)
    p = jnp.where(key >= thr32, jnp.exp(logits - m), 0.0)
    return p, 1.0 / jnp.sum(p, axis=0, keepdims=True)


def _gating_kernel(vals_ref, alpha_ref, temp_ref, u_ref, v_ref, lnw_ref,
                   lnb_ref, out_ref, attn_alpha_scr, *, k, scale):
    h = pl.program_id(0)
    b = pl.program_id(1)
    vals = vals_ref[0]

    energy = jnp.mean(vals * vals, axis=1, keepdims=True)
    rms = jnp.maximum(jnp.sqrt(jnp.mean(energy)), 1e-6)
    gain = jax.nn.softplus(temp_ref[h, 0])
    score = energy * (gain / rms)
    mu = jnp.mean(score)
    var = jnp.mean((score - mu) ** 2)
    score = (score - mu) * jax.lax.rsqrt(var + 1e-5)
    score = score * lnw_ref[:, :] + lnb_ref[:, :]
    bilinear = jnp.dot(v_ref[0], u_ref[0],
                       preferred_element_type=jnp.float32)
    logits = bilinear + score

    p, rz = _topk_softmax_t(logits, k)

    @pl.when(b == 0)
    def _():
        alpha_t = jnp.transpose(alpha_ref[0])
        pa, rza = _topk_softmax_t(alpha_t * scale, k)
        attn_alpha_scr[:, :] = pa * rza

    attn = p * rz + attn_alpha_scr[:, :]
    out_ref[0] = jax.lax.dot_general(
        attn, vals, (((0,), (0,)), ((), ())),
        preferred_element_type=jnp.float32)


def kernel(values, alpha, temp, gamma_hs, U, V, ln_w, ln_b):
    del gamma_hs
    B, F, H, D = values.shape
    _, S, _ = alpha.shape
    R = U.shape[-1]
    k = max(1, int(0.1 * F))
    scale = 1.0 / (F ** 0.5)

    vals2 = values.reshape(B, F, H * D)
    u_t = jnp.transpose(U, (0, 2, 1))
    v_t = jnp.transpose(V, (0, 2, 1))

    grid = (H, B)
    out = pl.pallas_call(
        functools.partial(_gating_kernel, k=k, scale=scale),
        grid=grid,
        in_specs=[
            pl.BlockSpec((1, F, D), lambda h, b: (b, 0, h)),
            pl.BlockSpec((1, S, F), lambda h, b: (h, 0, 0)),
            pl.BlockSpec(memory_space=pltpu.SMEM),
            pl.BlockSpec((1, R, S), lambda h, b: (h, 0, 0)),
            pl.BlockSpec((1, F, R), lambda h, b: (h, 0, 0)),
            pl.BlockSpec((F, 1), lambda h, b: (0, 0)),
            pl.BlockSpec((F, 1), lambda h, b: (0, 0)),
        ],
        out_specs=pl.BlockSpec((1, S, D), lambda h, b: (b, 0, h)),
        out_shape=jax.ShapeDtypeStruct((B, S, H * D), jnp.float32),
        scratch_shapes=[pltpu.VMEM((F, S), jnp.float32)],
        compiler_params=pltpu.CompilerParams(
            dimension_semantics=("arbitrary", "arbitrary"),
        ),
    )(vals2, alpha, temp.astype(jnp.float32), u_t, v_t,
      ln_w.reshape(F, 1), ln_b.reshape(F, 1))
    return out.reshape(B, S, H, D)


# speedup vs baseline: 1.4236x; 1.0125x over previous
import functools

import jax
import jax.numpy as jnp
from jax.experimental import pallas as pl
from jax.experimental.pallas import tpu as pltpu

_SIGN = -2147483648


def _count_ge16(key16, thr):
    f, cols = key16.shape
    one = jnp.int16(1)
    zero = jnp.int16(0)
    acc = jnp.zeros((16, cols), jnp.int16)
    for j in range(f // 16):
        acc = acc + jnp.where(key16[16 * j:16 * (j + 1)] >= thr, one, zero)
    return jnp.sum(acc.astype(jnp.int32), axis=0, keepdims=True)


def _count_gt16(key16, thr):
    f, cols = key16.shape
    one = jnp.int16(1)
    zero = jnp.int16(0)
    acc = jnp.zeros((16, cols), jnp.int16)
    for j in range(f // 16):
        acc = acc + jnp.where(key16[16 * j:16 * (j + 1)] > thr, one, zero)
    return jnp.sum(acc.astype(jnp.int32), axis=0, keepdims=True)


def _radix16(key16, target, cols):
    def body(i, t):
        bit = jnp.left_shift(jnp.int32(1), 15 - i)
        t_try = t | bit
        thr = (t_try ^ 0x8000).astype(jnp.int16)
        cnt = _count_ge16(key16, thr)
        return jnp.where(cnt >= target, t_try, t)

    return jax.lax.fori_loop(0, 16, body, jnp.zeros((1, cols), jnp.int32))


def _topk_softmax_t(logits, k):
    bits = jax.lax.bitcast_convert_type(logits, jnp.int32)
    key = bits ^ jnp.where(bits < 0, jnp.int32(0x7FFFFFFF), jnp.int32(0))
    khi = jax.lax.shift_right_arithmetic(key, 16).astype(jnp.int16)
    klo = (key.astype(jnp.int16)) ^ jnp.int16(-32768)
    cols = logits.shape[1]

    t_a = _radix16(khi, jnp.int32(k), cols)
    t_hi = (t_a ^ 0x8000).astype(jnp.int16)

    c_gt = _count_gt16(khi, t_hi)
    klo_m = jnp.where(khi == t_hi, klo, jnp.int16(-32768))
    t_b = _radix16(klo_m, jnp.int32(k) - c_gt, cols)

    thr32 = (jnp.left_shift(t_a, 16) | t_b) ^ jnp.int32(_SIGN)
    m = jnp.max(logits, axis=0, keepdims=True)
    p = jnp.where(key >= thr32, jnp.exp(logits - m), 0.0)
    return p, 1.0 / jnp.sum(p, axis=0, keepdims=True)


def _gating_kernel(vals_ref, alpha_ref, temp_ref, u_ref, v_ref, lnw_ref,
                   lnb_ref, out_ref, attn_alpha_scr, *, k, scale):
    h = pl.program_id(0)
    b = pl.program_id(1)
    vals = vals_ref[0]

    f_dim, d_dim = vals.shape
    sq = vals * vals
    energy = jnp.dot(sq, jnp.full((d_dim, 1), 1.0 / d_dim, jnp.float32),
                     preferred_element_type=jnp.float32)
    rms = jnp.maximum(jnp.sqrt(jnp.mean(energy)), 1e-6)
    gain = jax.nn.softplus(temp_ref[h, 0])
    score = energy * (gain / rms)
    mu = jnp.mean(score)
    var = jnp.mean((score - mu) ** 2)
    score = (score - mu) * jax.lax.rsqrt(var + 1e-5)
    score = score * lnw_ref[:, :] + lnb_ref[:, :]
    s_dim = u_ref.shape[-1]
    lhs = jnp.concatenate([v_ref[0], score], axis=1)
    rhs = jnp.concatenate(
        [u_ref[0], jnp.ones((1, s_dim), jnp.float32)], axis=0)
    logits = jnp.dot(lhs, rhs, preferred_element_type=jnp.float32)

    p, rz = _topk_softmax_t(logits, k)

    @pl.when(b == 0)
    def _():
        alpha_t = jnp.transpose(alpha_ref[0])
        pa, rza = _topk_softmax_t(alpha_t * scale, k)
        attn_alpha_scr[:, :] = pa * rza

    attn = p * rz + attn_alpha_scr[:, :]
    out_ref[0] = jax.lax.dot_general(
        attn, vals, (((0,), (0,)), ((), ())),
        preferred_element_type=jnp.float32)


def kernel(values, alpha, temp, gamma_hs, U, V, ln_w, ln_b):
    del gamma_hs
    B, F, H, D = values.shape
    _, S, _ = alpha.shape
    R = U.shape[-1]
    k = max(1, int(0.1 * F))
    scale = 1.0 / (F ** 0.5)

    vals2 = values.reshape(B, F, H * D)
    u_t = jnp.transpose(U, (0, 2, 1))
    v_t = jnp.transpose(V, (0, 2, 1))

    grid = (H, B)
    out = pl.pallas_call(
        functools.partial(_gating_kernel, k=k, scale=scale),
        grid=grid,
        in_specs=[
            pl.BlockSpec((1, F, D), lambda h, b: (b, 0, h)),
            pl.BlockSpec((1, S, F), lambda h, b: (h, 0, 0)),
            pl.BlockSpec(memory_space=pltpu.SMEM),
            pl.BlockSpec((1, R, S), lambda h, b: (h, 0, 0)),
            pl.BlockSpec((1, F, R), lambda h, b: (h, 0, 0)),
            pl.BlockSpec((F, 1), lambda h, b: (0, 0)),
            pl.BlockSpec((F, 1), lambda h, b: (0, 0)),
        ],
        out_specs=pl.BlockSpec((1, S, D), lambda h, b: (b, 0, h)),
        out_shape=jax.ShapeDtypeStruct((B, S, H * D), jnp.float32),
        scratch_shapes=[pltpu.VMEM((F, S), jnp.float32)],
        compiler_params=pltpu.CompilerParams(
            dimension_semantics=("arbitrary", "arbitrary"),
        ),
    )(vals2, alpha, temp.astype(jnp.float32), u_t, v_t,
      ln_w.reshape(F, 1), ln_b.reshape(F, 1))
    return out.reshape(B, S, H, D)


# speedup vs baseline: 1.4546x; 1.0218x over previous
import functools

import jax
import jax.numpy as jnp
from jax.experimental import pallas as pl
from jax.experimental.pallas import tpu as pltpu

_SIGN = -2147483648


def _count_ge16(key16, thr):
    f, cols = key16.shape
    one = jnp.int16(1)
    zero = jnp.int16(0)
    acc = jnp.zeros((16, cols), jnp.int16)
    for j in range(f // 16):
        acc = acc + jnp.where(key16[16 * j:16 * (j + 1)] >= thr, one, zero)
    return jnp.sum(acc.astype(jnp.int32), axis=0, keepdims=True)


def _count_gt16(key16, thr):
    f, cols = key16.shape
    one = jnp.int16(1)
    zero = jnp.int16(0)
    acc = jnp.zeros((16, cols), jnp.int16)
    for j in range(f // 16):
        acc = acc + jnp.where(key16[16 * j:16 * (j + 1)] > thr, one, zero)
    return jnp.sum(acc.astype(jnp.int32), axis=0, keepdims=True)


def _radix16(key16, target, cols):
    def body(i, t):
        bit = jnp.left_shift(jnp.int32(1), 15 - i)
        t_try = t | bit
        thr = (t_try ^ 0x8000).astype(jnp.int16)
        cnt = _count_ge16(key16, thr)
        return jnp.where(cnt >= target, t_try, t)

    return jax.lax.fori_loop(0, 16, body, jnp.zeros((1, cols), jnp.int32))


def _topk_softmax_t(logits, k):
    bits = jax.lax.bitcast_convert_type(logits, jnp.int32)
    key = bits ^ jnp.where(bits < 0, jnp.int32(0x7FFFFFFF), jnp.int32(0))
    khi = jax.lax.shift_right_arithmetic(key, 16).astype(jnp.int16)
    klo = (key.astype(jnp.int16)) ^ jnp.int16(-32768)
    cols = logits.shape[1]

    t_a = _radix16(khi, jnp.int32(k), cols)
    t_hi = (t_a ^ 0x8000).astype(jnp.int16)

    c_gt = _count_gt16(khi, t_hi)
    klo_m = jnp.where(khi == t_hi, klo, jnp.int16(-32768))
    t_b = _radix16(klo_m, jnp.int32(k) - c_gt, cols)

    thr32 = (jnp.left_shift(t_a, 16) | t_b) ^ jnp.int32(_SIGN)
    m = jnp.max(logits, axis=0, keepdims=True)
    p = jnp.where(key >= thr32, jnp.exp(logits - m), 0.0)
    return p, 1.0 / jnp.sum(p, axis=0, keepdims=True)


def _gating_kernel(vals_ref, alpha_ref, temp_ref, u_ref, v_ref, lnw_ref,
                   lnb_ref, out_ref, attn_alpha_scr, *, k, scale):
    h = pl.program_id(0)
    b = pl.program_id(1)
    vals = vals_ref[0]

    energy = jnp.mean(vals * vals, axis=1, keepdims=True)
    rms = jnp.maximum(jnp.sqrt(jnp.mean(energy)), 1e-6)
    gain = jax.nn.softplus(temp_ref[h, 0])
    score = energy * (gain / rms)
    mu = jnp.mean(score)
    var = jnp.mean((score - mu) ** 2)
    score = (score - mu) * jax.lax.rsqrt(var + 1e-5)
    score = score * lnw_ref[:, :] + lnb_ref[:, :]
    bilinear = jnp.dot(v_ref[0], u_ref[0],
                       preferred_element_type=jnp.float32)
    logits = bilinear + score

    p, rz = _topk_softmax_t(logits, k)

    @pl.when(b == 0)
    def _():
        alpha_t = jnp.transpose(alpha_ref[0])
        pa, rza = _topk_softmax_t(alpha_t * scale, k)
        attn_alpha_scr[:, :] = pa * rza

    attn = p * rz + attn_alpha_scr[:, :]
    out_ref[0] = jax.lax.dot_general(
        attn, vals, (((0,), (0,)), ((), ())),
        preferred_element_type=jnp.float32)


def kernel(values, alpha, temp, gamma_hs, U, V, ln_w, ln_b):
    del gamma_hs
    B, F, H, D = values.shape
    _, S, _ = alpha.shape
    R = U.shape[-1]
    k = max(1, int(0.1 * F))
    scale = 1.0 / (F ** 0.5)

    vals2 = values.reshape(B, F, H * D)
    u_t = jnp.transpose(U, (0, 2, 1))
    v_t = jnp.transpose(V, (0, 2, 1))

    grid = (H, B)
    out = pl.pallas_call(
        functools.partial(_gating_kernel, k=k, scale=scale),
        grid=grid,
        in_specs=[
            pl.BlockSpec((1, F, D), lambda h, b: (b, 0, h)),
            pl.BlockSpec((1, S, F), lambda h, b: (h, 0, 0)),
            pl.BlockSpec(memory_space=pltpu.SMEM),
            pl.BlockSpec((1, R, S), lambda h, b: (h, 0, 0)),
            pl.BlockSpec((1, F, R), lambda h, b: (h, 0, 0)),
            pl.BlockSpec((F, 1), lambda h, b: (0, 0)),
            pl.BlockSpec((F, 1), lambda h, b: (0, 0)),
        ],
        out_specs=pl.BlockSpec((1, S, D), lambda h, b: (b, 0, h)),
        out_shape=jax.ShapeDtypeStruct((B, S, H * D), jnp.float32),
        scratch_shapes=[pltpu.VMEM((F, S), jnp.float32)],
        compiler_params=pltpu.CompilerParams(
            dimension_semantics=("arbitrary", "arbitrary"),
        ),
    )(vals2, alpha, temp.astype(jnp.float32), u_t, v_t,
      ln_w.reshape(F, 1), ln_b.reshape(F, 1))
    return out.reshape(B, S, H, D)


# speedup vs baseline: 1.4657x; 1.0076x over previous
import functools

import jax
import jax.numpy as jnp
from jax.experimental import pallas as pl
from jax.experimental.pallas import tpu as pltpu

_SIGN = -2147483648


def _count_ge16(key16, thr):
    f, cols = key16.shape
    one = jnp.int16(1)
    zero = jnp.int16(0)
    acc = jnp.zeros((16, cols), jnp.int16)
    for j in range(f // 16):
        acc = acc + jnp.where(key16[16 * j:16 * (j + 1)] >= thr, one, zero)
    return jnp.sum(acc.astype(jnp.int32), axis=0, keepdims=True)


def _count_gt16(key16, thr):
    f, cols = key16.shape
    one = jnp.int16(1)
    zero = jnp.int16(0)
    acc = jnp.zeros((16, cols), jnp.int16)
    for j in range(f // 16):
        acc = acc + jnp.where(key16[16 * j:16 * (j + 1)] > thr, one, zero)
    return jnp.sum(acc.astype(jnp.int32), axis=0, keepdims=True)


def _radix16(key16, target, cols):
    def body(i, t):
        bit = jnp.left_shift(jnp.int32(1), 15 - i)
        t_try = t | bit
        thr = (t_try ^ 0x8000).astype(jnp.int16)
        cnt = _count_ge16(key16, thr)
        return jnp.where(cnt >= target, t_try, t)

    return jax.lax.fori_loop(0, 16, body, jnp.zeros((1, cols), jnp.int32))


def _topk_softmax_t(logits, k):
    bits = jax.lax.bitcast_convert_type(logits, jnp.int32)
    key = bits ^ jnp.where(bits < 0, jnp.int32(0x7FFFFFFF), jnp.int32(0))
    khi = jax.lax.shift_right_arithmetic(key, 16).astype(jnp.int16)
    klo = (key.astype(jnp.int16)) ^ jnp.int16(-32768)
    cols = logits.shape[1]

    t_a = _radix16(khi, jnp.int32(k), cols)
    t_hi = (t_a ^ 0x8000).astype(jnp.int16)

    c_gt = _count_gt16(khi, t_hi)
    klo_m = jnp.where(khi == t_hi, klo, jnp.int16(-32768))
    t_b = _radix16(klo_m, jnp.int32(k) - c_gt, cols)

    thr32 = (jnp.left_shift(t_a, 16) | t_b) ^ jnp.int32(_SIGN)
    m = jnp.max(logits, axis=0, keepdims=True)
    p = jnp.where(key >= thr32, jnp.exp(logits - m), 0.0)
    return p, 1.0 / jnp.sum(p, axis=0, keepdims=True)


def _gating_kernel(vals_ref, alpha_ref, temp_ref, u_ref, v_ref, lnw_ref,
                   lnb_ref, out_ref, attn_alpha_scr, *, k, scale):
    h = pl.program_id(0)
    b = pl.program_id(1)
    vals = vals_ref[0]

    energy = jnp.mean(vals * vals, axis=1, keepdims=True)
    rms = jnp.maximum(jnp.sqrt(jnp.mean(energy)), 1e-6)
    gain = jax.nn.softplus(temp_ref[h, 0])
    score = energy * (gain / rms)
    mu = jnp.mean(score)
    var = jnp.mean((score - mu) ** 2)
    score = (score - mu) * jax.lax.rsqrt(var + 1e-5)
    score = score * lnw_ref[:, :] + lnb_ref[:, :]
    bilinear = jnp.dot(v_ref[0], u_ref[0],
                       preferred_element_type=jnp.float32)
    logits = bilinear + score

    p, rz = _topk_softmax_t(logits, k)

    @pl.when(b == 0)
    def _():
        alpha_t = jnp.transpose(alpha_ref[0])
        pa, rza = _topk_softmax_t(alpha_t * scale, k)
        attn_alpha_scr[:, :] = pa * rza

    attn = (p * rz + attn_alpha_scr[:, :]).astype(jnp.bfloat16)
    out_ref[0] = jax.lax.dot_general(
        attn, vals.astype(jnp.bfloat16), (((0,), (0,)), ((), ())),
        preferred_element_type=jnp.float32)


def kernel(values, alpha, temp, gamma_hs, U, V, ln_w, ln_b):
    del gamma_hs
    B, F, H, D = values.shape
    _, S, _ = alpha.shape
    R = U.shape[-1]
    k = max(1, int(0.1 * F))
    scale = 1.0 / (F ** 0.5)

    vals2 = values.reshape(B, F, H * D)
    u_t = jnp.transpose(U, (0, 2, 1))
    v_t = jnp.transpose(V, (0, 2, 1))

    grid = (H, B)
    out = pl.pallas_call(
        functools.partial(_gating_kernel, k=k, scale=scale),
        grid=grid,
        in_specs=[
            pl.BlockSpec((1, F, D), lambda h, b: (b, 0, h)),
            pl.BlockSpec((1, S, F), lambda h, b: (h, 0, 0)),
            pl.BlockSpec(memory_space=pltpu.SMEM),
            pl.BlockSpec((1, R, S), lambda h, b: (h, 0, 0)),
            pl.BlockSpec((1, F, R), lambda h, b: (h, 0, 0)),
            pl.BlockSpec((F, 1), lambda h, b: (0, 0)),
            pl.BlockSpec((F, 1), lambda h, b: (0, 0)),
        ],
        out_specs=pl.BlockSpec((1, S, D), lambda h, b: (b, 0, h)),
        out_shape=jax.ShapeDtypeStruct((B, S, H * D), jnp.float32),
        scratch_shapes=[pltpu.VMEM((F, S), jnp.float32)],
        compiler_params=pltpu.CompilerParams(
            dimension_semantics=("arbitrary", "arbitrary"),
        ),
    )(vals2, alpha, temp.astype(jnp.float32), u_t, v_t,
      ln_w.reshape(F, 1), ln_b.reshape(F, 1))
    return out.reshape(B, S, H, D)
